# baseline (speedup 1.0000x reference)
import functools
import math

import jax
import jax.numpy as jnp
from jax.experimental import pallas as pl
from jax.experimental.pallas import tpu as pltpu

HID = 128
NL = 4
NRBF = 20
K = 10
CUTOFF = 5.0
NTYPES = 10
BB = 16


def _bdot(a, b):
    return jnp.dot(a.astype(jnp.bfloat16), b.astype(jnp.bfloat16),
                   preferred_element_type=jnp.float32)


def _xdot(a, b):
    return jnp.dot(a, b, preferred_element_type=jnp.float32,
                   precision=jax.lax.Precision.HIGHEST)


def _silu(x):
    return x * jax.nn.sigmoid(x)


def _net_kernel(posc_ref, posf_ref, atnum_ref, cent_ref, eye_ref, embed_ref,
                posW_ref, posb_ref,
                mW1_ref, mb1_ref, mW2_ref, mb2_ref, uW_ref, ub_ref,
                gWs_ref, gbs_ref, gWm_ref, gbm_ref, fW_ref, fb_ref,
                gam_ref, bet_ref, cW1_ref, cb1_ref, cW2_ref, cb2_ref,
                oW1_ref, ob1_ref, oW2_ref, ob2_ref, out_ref, *, n_atoms):
    f32 = jnp.float32
    bf16 = jnp.bfloat16
    N = n_atoms
    R = BB * N

    posc = posc_ref[...]
    posf = posf_ref[...]
    d2 = None
    for c in range(3):
        xs = posf[:, c:c + 1].reshape(BB, N, 1)
        dc = xs - posc[c][:, None, :]
        d2 = dc * dc if d2 is None else d2 + dc * dc
    dist = jnp.sqrt(d2 + 1e-12)
    d0 = (dist + eye_ref[...]).reshape(R, N)

    a_lane = d0[:, None, :]
    b_sub = d0[:, :, None]
    i_sub = jax.lax.broadcasted_iota(jnp.int32, (R, N, N), 1)
    i_lane = jax.lax.broadcasted_iota(jnp.int32, (R, N, N), 2)
    before = (b_sub < a_lane) | ((b_sub == a_lane) & (i_sub < i_lane))
    rank = jnp.sum(before.astype(jnp.int32), axis=1)

    lane_i = jax.lax.broadcasted_iota(jnp.int32, (R, N), 1)
    lane_f = lane_i.astype(f32)
    bidx_node = jax.lax.broadcasted_iota(jnp.int32, (BB, N, 1), 0).reshape(R, 1)
    lane_g = jax.lax.broadcasted_iota(jnp.int32, (R, R), 1)
    width = CUTOFF / NRBF
    rbf_inv = 1.0 / (2.0 * width * width)
    centers = cent_ref[...]

    onehots, rbfs, envs = [], [], []
    envsum = None
    for k in range(K):
        sel = rank == k
        mval = jnp.min(jnp.where(sel, d0, f32(jnp.inf)), axis=1, keepdims=True)
        midx = jnp.min(jnp.where(sel, lane_f, f32(1e9)), axis=1, keepdims=True)
        gi_k = midx.astype(jnp.int32) + bidx_node * N
        onehots.append((lane_g == gi_k).astype(bf16))
        rbfs.append(jnp.exp(-((mval - centers) ** 2) * rbf_inv).astype(bf16))
        env_k = jnp.where(mval < CUTOFF,
                          0.5 * (jnp.cos(mval * (math.pi / CUTOFF)) + 1.0),
                          f32(0.0))
        envs.append(jnp.broadcast_to(env_k, (R, HID)))
        envsum = env_k if envsum is None else envsum + env_k

    at = atnum_ref[...]
    lane_t = jax.lax.broadcasted_iota(jnp.int32, (R, NTYPES), 1)
    oh_at = (lane_t == at).astype(f32)
    af = _xdot(oh_at, embed_ref[...])
    pf = _bdot(posf_ref[...], posW_ref[...]) + posb_ref[...]
    h = jnp.concatenate([af, pf], axis=1)

    for l in range(NL):
        W1 = mW1_ref[l]
        W1b = W1[HID:].astype(bf16)
        hb = h.astype(bf16)
        ci = jnp.dot(hb, W1[:HID].astype(bf16),
                     preferred_element_type=f32) + mb1_ref[l]
        s = None
        for k in range(K):
            hjk = jnp.dot(onehots[k], hb, preferred_element_type=f32)
            mk = jnp.dot(jnp.concatenate([hjk.astype(bf16), rbfs[k]], axis=1),
                         W1b, preferred_element_type=f32) + ci
            mk = _silu(mk)
            mk = mk.astype(bf16).astype(f32) * envs[k]
            s = mk if s is None else s + mk
        W2r = mW2_ref[l].astype(bf16).astype(f32)
        agg = _xdot(s, W2r) + envsum * mb2_ref[l]

        hu = jnp.concatenate([h, agg], axis=1)
        local = _silu(_bdot(hu, uW_ref[l]) + ub_ref[l])

        mean_h = jnp.mean(h.reshape(BB, N, HID), axis=1)
        gterm = _bdot(mean_h, gWm_ref[l]) + gbm_ref[l]
        gterm_e = jnp.broadcast_to(gterm[:, None, :], (BB, N, HID)).reshape(R, HID)
        glob = _silu(_bdot(h, gWs_ref[l]) + gbs_ref[l] + gterm_e)

        fused = _silu(_bdot(jnp.concatenate([local, glob], axis=1), fW_ref[l])
                      + fb_ref[l])

        mu = jnp.mean(fused, axis=1, keepdims=True)
        var = jnp.mean((fused - mu) ** 2, axis=1, keepdims=True)
        zn = (fused - mu) / jnp.sqrt(var + 1e-5) * gam_ref[l] + bet_ref[l]
        zz = _silu(_bdot(zn, cW1_ref[l]) + cb1_ref[l])
        h = fused + _bdot(zz, cW2_ref[l]) + cb2_ref[l]

    o = _silu(_bdot(h, oW1_ref[...]) + ob1_ref[...])
    out_ref[...] = _bdot(o, oW2_ref[...]) + ob2_ref[...]


def kernel(positions, atomic_numbers, atom_embed, pos_W, pos_b,
           msg_W1, msg_b1, msg_W2, msg_b2, upd_W, upd_b,
           glob_Ws, glob_bs, glob_Wm, glob_bm, fus_W, fus_b,
           cb_gamma, cb_beta, cb_W1, cb_b1, cb_W2, cb_b2,
           out_W1, out_b1, out_W2, out_b2):
    B, N, _ = positions.shape
    assert B % BB == 0
    R = BB * N

    pos_coords = jnp.transpose(positions, (2, 0, 1))
    pos_flat = positions.reshape(B * N, 3)
    atnum = atomic_numbers.reshape(B * N, 1).astype(jnp.int32)
    centers = jnp.linspace(0.0, CUTOFF, NRBF).reshape(1, NRBF).astype(jnp.float32)
    eye = (jnp.eye(N, dtype=jnp.float32) * 1e6).reshape(1, N, N)

    const = lambda *shape: pl.BlockSpec(shape, lambda i: (0,) * len(shape))
    in_specs = [
        pl.BlockSpec((3, BB, N), lambda i: (0, i, 0)),
        pl.BlockSpec((R, 3), lambda i: (i, 0)),
        pl.BlockSpec((R, 1), lambda i: (i, 0)),
        const(1, NRBF),
        const(1, N, N),
        const(NTYPES, HID // 4),
        const(3, HID * 3 // 4),
        const(1, HID * 3 // 4),
        const(NL, 2 * HID + NRBF, HID),
        const(NL, 1, HID),
        const(NL, HID, HID),
        const(NL, 1, HID),
        const(NL, 2 * HID, HID),
        const(NL, 1, HID),
        const(NL, HID, HID),
        const(NL, 1, HID),
        const(NL, HID, HID),
        const(NL, 1, HID),
        const(NL, 2 * HID, HID),
        const(NL, 1, HID),
        const(NL, 1, HID),
        const(NL, 1, HID),
        const(NL, HID, HID),
        const(NL, 1, HID),
        const(NL, HID, HID),
        const(NL, 1, HID),
        const(HID, HID),
        const(1, HID),
        const(HID, 3),
        const(1, 3),
    ]

    out = pl.pallas_call(
        functools.partial(_net_kernel, n_atoms=N),
        grid=(B // BB,),
        in_specs=in_specs,
        out_specs=pl.BlockSpec((R, 3), lambda i: (i, 0)),
        out_shape=jax.ShapeDtypeStruct((B * N, 3), jnp.float32),
        compiler_params=pltpu.CompilerParams(
            dimension_semantics=("parallel",)),
    )(pos_coords, pos_flat, atnum, centers, eye, atom_embed, pos_W,
      pos_b.reshape(1, -1),
      msg_W1, msg_b1.reshape(NL, 1, HID), msg_W2, msg_b2.reshape(NL, 1, HID),
      upd_W, upd_b.reshape(NL, 1, HID),
      glob_Ws, glob_bs.reshape(NL, 1, HID), glob_Wm, glob_bm.reshape(NL, 1, HID),
      fus_W, fus_b.reshape(NL, 1, HID),
      cb_gamma.reshape(NL, 1, HID), cb_beta.reshape(NL, 1, HID),
      cb_W1, cb_b1.reshape(NL, 1, HID), cb_W2, cb_b2.reshape(NL, 1, HID),
      out_W1, out_b1.reshape(1, -1), out_W2, out_b2.reshape(1, -1))
    return out.reshape(B, N, 3)


# speedup vs baseline: 1.1232x; 1.1232x over previous
import functools
import math

import jax
import jax.numpy as jnp
from jax.experimental import pallas as pl
from jax.experimental.pallas import tpu as pltpu

HID = 128
NL = 4
NRBF = 20
K = 10
CUTOFF = 5.0
NTYPES = 10
BB = 8


def _bdot(a, b):
    return jnp.dot(a.astype(jnp.bfloat16), b.astype(jnp.bfloat16),
                   preferred_element_type=jnp.float32)


def _xdot(a, b):
    return jnp.dot(a, b, preferred_element_type=jnp.float32,
                   precision=jax.lax.Precision.HIGHEST)


def _silu(x):
    return x * jax.nn.sigmoid(x)


def _net_kernel(posc_ref, posf_ref, atnum_ref, cent_ref, eye_ref, embed_ref,
                posW_ref, posb_ref,
                mW1_ref, mb1_ref, mW2_ref, mb2_ref, uW_ref, ub_ref,
                gWs_ref, gbs_ref, gWm_ref, gbm_ref, fW_ref, fb_ref,
                gam_ref, bet_ref, cW1_ref, cb1_ref, cW2_ref, cb2_ref,
                oW1_ref, ob1_ref, oW2_ref, ob2_ref, out_ref, *, n_atoms):
    f32 = jnp.float32
    bf16 = jnp.bfloat16
    N = n_atoms
    R = BB * N

    posc = posc_ref[...]
    posf = posf_ref[...]
    d2 = None
    for c in range(3):
        xs = posf[:, c:c + 1].reshape(BB, N, 1)
        dc = xs - posc[c][:, None, :]
        d2 = dc * dc if d2 is None else d2 + dc * dc
    dist = jnp.sqrt(d2 + 1e-12)
    d0 = (dist + eye_ref[...]).reshape(R, N)

    a_lane = d0[:, None, :]
    b_sub = d0[:, :, None]
    i_sub = jax.lax.broadcasted_iota(jnp.int32, (R, N, N), 1)
    i_lane = jax.lax.broadcasted_iota(jnp.int32, (R, N, N), 2)
    before = (b_sub < a_lane) | ((b_sub == a_lane) & (i_sub < i_lane))
    rank = jnp.sum(before.astype(jnp.int32), axis=1)

    width = CUTOFF / NRBF
    rbf_inv = 1.0 / (2.0 * width * width)
    centers = cent_ref[...]

    bi0 = jax.lax.broadcasted_iota(jnp.int32, (BB, N, BB, N), 0)
    bi2 = jax.lax.broadcasted_iota(jnp.int32, (BB, N, BB, N), 2)
    rankg = jnp.where(bi0 == bi2,
                      jnp.broadcast_to(rank.reshape(BB, N, 1, N),
                                       (BB, N, BB, N)),
                      jnp.int32(99)).reshape(R, R)

    onehots, rbfs, envs = [], [], []
    envsum = None
    for k in range(K):
        sel = rank == k
        mval = jnp.min(jnp.where(sel, d0, f32(jnp.inf)), axis=1, keepdims=True)
        onehots.append((rankg == k).astype(bf16))
        rbfs.append(jnp.exp(-((mval - centers) ** 2) * rbf_inv).astype(bf16))
        env_k = jnp.where(mval < CUTOFF,
                          0.5 * (jnp.cos(mval * (math.pi / CUTOFF)) + 1.0),
                          f32(0.0))
        envs.append(jnp.broadcast_to(env_k, (R, HID)))
        envsum = env_k if envsum is None else envsum + env_k

    at = atnum_ref[...]
    lane_t = jax.lax.broadcasted_iota(jnp.int32, (R, NTYPES), 1)
    oh_at = (lane_t == at).astype(f32)
    af = _xdot(oh_at, embed_ref[...])
    pf = _bdot(posf_ref[...], posW_ref[...]) + posb_ref[...]
    h = jnp.concatenate([af, pf], axis=1)

    for l in range(NL):
        W1 = mW1_ref[l]
        W1b = W1[HID:].astype(bf16)
        hb = h.astype(bf16)
        ci = jnp.dot(hb, W1[:HID].astype(bf16),
                     preferred_element_type=f32) + mb1_ref[l]
        s = None
        for k in range(K):
            hjk = jnp.dot(onehots[k], hb, preferred_element_type=f32)
            mk = jnp.dot(jnp.concatenate([hjk.astype(bf16), rbfs[k]], axis=1),
                         W1b, preferred_element_type=f32) + ci
            mk = _silu(mk)
            mk = mk.astype(bf16).astype(f32) * envs[k]
            s = mk if s is None else s + mk
        W2r = mW2_ref[l].astype(bf16).astype(f32)
        agg = _xdot(s, W2r) + envsum * mb2_ref[l]

        hu = jnp.concatenate([h, agg], axis=1)
        local = _silu(_bdot(hu, uW_ref[l]) + ub_ref[l])

        mean_h = jnp.mean(h.reshape(BB, N, HID), axis=1)
        gterm = _bdot(mean_h, gWm_ref[l]) + gbm_ref[l]
        gterm_e = jnp.broadcast_to(gterm[:, None, :], (BB, N, HID)).reshape(R, HID)
        glob = _silu(_bdot(h, gWs_ref[l]) + gbs_ref[l] + gterm_e)

        fused = _silu(_bdot(jnp.concatenate([local, glob], axis=1), fW_ref[l])
                      + fb_ref[l])

        mu = jnp.mean(fused, axis=1, keepdims=True)
        var = jnp.mean((fused - mu) ** 2, axis=1, keepdims=True)
        zn = (fused - mu) / jnp.sqrt(var + 1e-5) * gam_ref[l] + bet_ref[l]
        zz = _silu(_bdot(zn, cW1_ref[l]) + cb1_ref[l])
        h = fused + _bdot(zz, cW2_ref[l]) + cb2_ref[l]

    o = _silu(_bdot(h, oW1_ref[...]) + ob1_ref[...])
    out_ref[...] = _bdot(o, oW2_ref[...]) + ob2_ref[...]


def kernel(positions, atomic_numbers, atom_embed, pos_W, pos_b,
           msg_W1, msg_b1, msg_W2, msg_b2, upd_W, upd_b,
           glob_Ws, glob_bs, glob_Wm, glob_bm, fus_W, fus_b,
           cb_gamma, cb_beta, cb_W1, cb_b1, cb_W2, cb_b2,
           out_W1, out_b1, out_W2, out_b2):
    B, N, _ = positions.shape
    assert B % BB == 0
    R = BB * N

    pos_coords = jnp.transpose(positions, (2, 0, 1))
    pos_flat = positions.reshape(B * N, 3)
    atnum = atomic_numbers.reshape(B * N, 1).astype(jnp.int32)
    centers = jnp.linspace(0.0, CUTOFF, NRBF).reshape(1, NRBF).astype(jnp.float32)
    eye = (jnp.eye(N, dtype=jnp.float32) * 1e6).reshape(1, N, N)

    const = lambda *shape: pl.BlockSpec(shape, lambda i: (0,) * len(shape))
    in_specs = [
        pl.BlockSpec((3, BB, N), lambda i: (0, i, 0)),
        pl.BlockSpec((R, 3), lambda i: (i, 0)),
        pl.BlockSpec((R, 1), lambda i: (i, 0)),
        const(1, NRBF),
        const(1, N, N),
        const(NTYPES, HID // 4),
        const(3, HID * 3 // 4),
        const(1, HID * 3 // 4),
        const(NL, 2 * HID + NRBF, HID),
        const(NL, 1, HID),
        const(NL, HID, HID),
        const(NL, 1, HID),
        const(NL, 2 * HID, HID),
        const(NL, 1, HID),
        const(NL, HID, HID),
        const(NL, 1, HID),
        const(NL, HID, HID),
        const(NL, 1, HID),
        const(NL, 2 * HID, HID),
        const(NL, 1, HID),
        const(NL, 1, HID),
        const(NL, 1, HID),
        const(NL, HID, HID),
        const(NL, 1, HID),
        const(NL, HID, HID),
        const(NL, 1, HID),
        const(HID, HID),
        const(1, HID),
        const(HID, 3),
        const(1, 3),
    ]

    out = pl.pallas_call(
        functools.partial(_net_kernel, n_atoms=N),
        grid=(B // BB,),
        in_specs=in_specs,
        out_specs=pl.BlockSpec((R, 3), lambda i: (i, 0)),
        out_shape=jax.ShapeDtypeStruct((B * N, 3), jnp.float32),
        compiler_params=pltpu.CompilerParams(
            dimension_semantics=("parallel",)),
    )(pos_coords, pos_flat, atnum, centers, eye, atom_embed, pos_W,
      pos_b.reshape(1, -1),
      msg_W1, msg_b1.reshape(NL, 1, HID), msg_W2, msg_b2.reshape(NL, 1, HID),
      upd_W, upd_b.reshape(NL, 1, HID),
      glob_Ws, glob_bs.reshape(NL, 1, HID), glob_Wm, glob_bm.reshape(NL, 1, HID),
      fus_W, fus_b.reshape(NL, 1, HID),
      cb_gamma.reshape(NL, 1, HID), cb_beta.reshape(NL, 1, HID),
      cb_W1, cb_b1.reshape(NL, 1, HID), cb_W2, cb_b2.reshape(NL, 1, HID),
      out_W1, out_b1.reshape(1, -1), out_W2, out_b2.reshape(1, -1))
    return out.reshape(B, N, 3)
